# speedup vs baseline: 3.8198x; 3.8198x over previous
import functools

import jax
import jax.numpy as jnp
from jax import lax
from jax.experimental import pallas as pl
from jax.experimental.pallas import tpu as pltpu
from jax.experimental.pallas import tpu_sc as plsc

_NC = 2
_NS = 16
_NW = _NC * _NS

_B, _S, _D = 1024, 200, 128
_N = _B * _S
_PER_W = _N // _NW
_CHUNK = _S
_NCHUNK = _PER_W // _CHUNK
_DV = _D // 16


def _body(seq_hbm, table_hbm, type_hbm, pos_hbm, out_hbm,
          idx_v, rows_v, bias_v, type_v, sem):
    wid = lax.axis_index("s") * _NC + lax.axis_index("c")
    base = wid * _PER_W

    pltpu.sync_copy(pos_hbm.at[pl.ds(0, _CHUNK)], bias_v)
    pltpu.sync_copy(type_hbm, type_v)

    @pl.loop(0, _CHUNK)
    def _bias(r):
        for t in range(_DV):
            sl = pl.ds(t * 16, 16)
            bias_v[r, sl] = bias_v[r, sl] + type_v[0, sl]

    @pl.loop(0, _NCHUNK)
    def _chunk(k):
        start = base + k * _CHUNK
        pltpu.sync_copy(seq_hbm.at[pl.ds(start, _CHUNK)], idx_v)
        cp0 = pltpu.async_copy(table_hbm.at[idx_v.at[pl.ds(0, 128)]],
                               rows_v.at[pl.ds(0, 128)], sem)
        cp1 = pltpu.async_copy(table_hbm.at[idx_v.at[pl.ds(128, 72)]],
                               rows_v.at[pl.ds(128, 72)], sem)
        cp0.wait()
        cp1.wait()

        @pl.loop(0, _CHUNK)
        def _add(r):
            for t in range(_DV):
                sl = pl.ds(t * 16, 16)
                rows_v[r, sl] = rows_v[r, sl] + bias_v[r, sl]

        pltpu.sync_copy(rows_v, out_hbm.at[pl.ds(start, _CHUNK)])


@jax.jit
def kernel(sequence, token_table, type_table, pos_table):
    seq_flat = sequence.reshape(_N)
    mesh = plsc.VectorSubcoreMesh(core_axis_name="c", subcore_axis_name="s")
    out = pl.kernel(
        _body,
        out_type=jax.ShapeDtypeStruct((_N, _D), jnp.float32),
        mesh=mesh,
        scratch_types=[
            pltpu.VMEM((_CHUNK,), jnp.int32),
            pltpu.VMEM((_CHUNK, _D), jnp.float32),
            pltpu.VMEM((_CHUNK, _D), jnp.float32),
            pltpu.VMEM((1, _D), jnp.float32),
            pltpu.SemaphoreType.DMA,
        ],
    )(seq_flat, token_table, type_table, pos_table)
    return out.reshape(_B, _S, _D)


# speedup vs baseline: 7.2455x; 1.8968x over previous
import jax
import jax.numpy as jnp
from jax import lax
from jax.experimental import pallas as pl
from jax.experimental.pallas import tpu as pltpu
from jax.experimental.pallas import tpu_sc as plsc

_NC = 2
_NS = 16
_NW = _NC * _NS
_B, _S, _D = 1024, 200, 128
_N = _B * _S
_PER_W = _N // _NW
_C = _S
_NCH = _PER_W // _C
_DV = _D // 16
_R = 4


def _body(seq_hbm, table_hbm, type_hbm, pos_hbm, out_hbm,
          idx_all, bufs, type_v, bias_sh,
          gs0, gs1, gs2, gs3, ss0, ss1, ss2, ss3):
    gsems = (gs0, gs1, gs2, gs3)
    ssems = (ss0, ss1, ss2, ss3)
    sid = lax.axis_index("s")
    wid = sid * _NC + lax.axis_index("c")
    base = wid * _PER_W

    pltpu.sync_copy(seq_hbm.at[pl.ds(wid * _NCH, _NCH)], idx_all)

    @pl.when(sid == 0)
    def _mk_bias():
        pltpu.sync_copy(pos_hbm.at[pl.ds(0, _S)], bufs.at[0])
        pltpu.sync_copy(type_hbm, type_v)

        @pl.loop(0, _S)
        def _add_type(r):
            for t in range(_DV):
                sl = pl.ds(t * 16, 16)
                bufs[0, r, sl] = bufs[0, r, sl] + type_v[0, sl]

        pltpu.sync_copy(bufs.at[0], bias_sh)

    plsc.subcore_barrier()

    def prefill(b):
        pltpu.sync_copy(bias_sh, bufs.at[b])

    def start_gather(c, b):
        pltpu.async_copy(table_hbm.at[idx_all.at[c, pl.ds(0, 128)]],
                         bufs.at[b, pl.ds(0, 128)], gsems[b], add=True)
        pltpu.async_copy(table_hbm.at[idx_all.at[c, pl.ds(128, 72)]],
                         bufs.at[b, pl.ds(128, 72)], gsems[b], add=True)

    def wait_gather(b):
        pltpu.make_async_copy(table_hbm.at[idx_all.at[0, pl.ds(0, 128)]],
                              bufs.at[b, pl.ds(0, 128)], gsems[b]).wait()
        pltpu.make_async_copy(table_hbm.at[idx_all.at[0, pl.ds(128, 72)]],
                              bufs.at[b, pl.ds(128, 72)], gsems[b]).wait()

    def start_store(c, b):
        pltpu.async_copy(bufs.at[b], out_hbm.at[pl.ds(base + c * _C, _C)],
                         ssems[b])

    def wait_store(b):
        pltpu.make_async_copy(bufs.at[b], out_hbm.at[pl.ds(0, _C)],
                              ssems[b]).wait()

    for b in range(2):
        prefill(b)
        start_gather(b, b)

    @pl.loop(0, _NCH // _R)
    def _outer(i):
        for b in range(_R):
            k = i * _R + b
            wait_gather(b)
            start_store(k, b)
            c = k + 2
            b2 = (b + 2) % _R

            @pl.when(c < _NCH)
            def _prep():
                @pl.when(c >= _R)
                def _w():
                    wait_store(b2)
                prefill(b2)
                start_gather(c, b2)

    for b in range(_R):
        wait_store(b)


@jax.jit
def kernel(sequence, token_table, type_table, pos_table):
    seq2 = sequence.reshape(_N // _C, _C)
    mesh = plsc.VectorSubcoreMesh(core_axis_name="c", subcore_axis_name="s")
    out = pl.kernel(
        _body,
        out_type=jax.ShapeDtypeStruct((_N, _D), jnp.float32),
        mesh=mesh,
        scratch_types=[
            pltpu.VMEM((_NCH, _C), jnp.int32),
            pltpu.VMEM((_R, _C, _D), jnp.float32),
            pltpu.VMEM((1, _D), jnp.float32),
            pltpu.VMEM_SHARED((_C, _D), jnp.float32),
        ] + [pltpu.SemaphoreType.DMA] * 8,
    )(seq2, token_table, type_table, pos_table)
    return out.reshape(_B, _S, _D)
